# Initial kernel scaffold; baseline (speedup 1.0000x reference)
import jax
import jax.numpy as jnp
from jax.experimental import pallas as pl


def kernel(inputs, emb):
    raise NotImplementedError("write your pallas kernel here")


# speedup vs baseline: 1.3712x; 1.3712x over previous
import functools

import jax
import jax.numpy as jnp
from jax.experimental import pallas as pl
from jax.experimental.pallas import tpu as pltpu

A = 4
K = 1024
D = 64
B = 16
T = 576
N = B * T


def _vq_body(x_ref, w_ref, q_ref, idx_ref, loss_ref, perp_ref, counts_ref,
             loss_acc):
    a = pl.program_id(0)
    b = pl.program_id(1)

    @pl.when(b == 0)
    def _reset():
        counts_ref[...] = jnp.zeros_like(counts_ref)
        loss_acc[0] = 0.0

    x = x_ref[0, 0]
    w = w_ref[0]
    xt = x.T
    mm = jax.lax.dot_general(xt, w, (((1,), (1,)), ((), ())),
                             preferred_element_type=jnp.float32)
    sx = jnp.sum(xt * xt, axis=1, keepdims=True)
    sw = jnp.sum(w * w, axis=1)
    dist = (sx + sw[None, :]) - 2.0 * mm

    m = jnp.min(dist, axis=1, keepdims=True)
    lane = jax.lax.broadcasted_iota(jnp.int32, (T, K), 1)
    idx = jnp.min(jnp.where(dist == m, lane, K), axis=1, keepdims=True)

    oh = (lane == idx).astype(jnp.float32)
    q = jax.lax.dot_general(w, oh, (((0,), (1,)), ((), ())),
                            preferred_element_type=jnp.float32)
    q_ref[0, 0] = q
    idx_ref[0, 0] = idx

    counts_ref[...] += jnp.sum(oh, axis=0, keepdims=True)
    loss_acc[0] += jnp.sum(m)

    @pl.when(b == B - 1)
    def _finalize():
        loss_ref[a] = loss_acc[0]
        p = counts_ref[...] / N
        perp_ref[a] = jnp.exp(-jnp.sum(p * jnp.log(p + 1e-10)))


@functools.partial(jax.jit)
def _vq(xt, emb):
    return pl.pallas_call(
        _vq_body,
        grid=(A, B),
        in_specs=[
            pl.BlockSpec((1, 1, D, T), lambda a, b: (a, b, 0, 0)),
            pl.BlockSpec((1, K, D), lambda a, b: (a, 0, 0)),
        ],
        out_specs=[
            pl.BlockSpec((1, 1, D, T), lambda a, b: (a, b, 0, 0)),
            pl.BlockSpec((1, 1, T, 1), lambda a, b: (a, b, 0, 0)),
            pl.BlockSpec(memory_space=pltpu.SMEM),
            pl.BlockSpec(memory_space=pltpu.SMEM),
        ],
        out_shape=[
            jax.ShapeDtypeStruct((A, B, D, T), jnp.float32),
            jax.ShapeDtypeStruct((A, B, T, 1), jnp.int32),
            jax.ShapeDtypeStruct((A,), jnp.float32),
            jax.ShapeDtypeStruct((A,), jnp.float32),
        ],
        scratch_shapes=[
            pltpu.VMEM((1, K), jnp.float32),
            pltpu.SMEM((1,), jnp.float32),
        ],
    )(xt, emb)


def kernel(inputs, emb):
    xt = jnp.transpose(inputs, (2, 0, 1, 3))
    q_t, idx_t, loss_sums, perps = _vq(xt, emb)
    quantized = jnp.transpose(q_t, (1, 2, 0, 3))
    encoding_indices = jnp.transpose(idx_t, (1, 2, 0, 3)).reshape(N, A, 1)
    l = loss_sums / jnp.float32(N * D)
    q_loss = jnp.sum(l) / A
    e_loss = jnp.sum(0.25 * l) / A
    perplexity = jnp.sum(perps) / A
    return q_loss, e_loss, quantized, perplexity, encoding_indices


# speedup vs baseline: 1.4968x; 1.0916x over previous
import jax
import jax.numpy as jnp
from jax.experimental import pallas as pl
from jax.experimental.pallas import tpu as pltpu

A = 4
K = 1024
D = 64
B = 16
T = 576
N = B * T


def _vq_body(x_ref, w_ref, q_ref, idx_ref, loss_ref, perp_ref, counts_ref):
    b = pl.program_id(0)

    @pl.when(b == 0)
    def _reset():
        counts_ref[...] = jnp.zeros_like(counts_ref)
        for a in range(A):
            loss_ref[a] = 0.0

    for a in range(A):
        x = x_ref[0, :, T * a:T * (a + 1)]
        w = w_ref[a]
        xt = x.T
        mm = jax.lax.dot_general(xt, w, (((1,), (1,)), ((), ())),
                                 preferred_element_type=jnp.float32)
        sx = jnp.sum(xt * xt, axis=1, keepdims=True)
        sw = jnp.sum(w * w, axis=1)
        dist = (sx + sw[None, :]) - 2.0 * mm

        m = jnp.min(dist, axis=1, keepdims=True)
        lane = jax.lax.broadcasted_iota(jnp.int32, (T, K), 1)
        idx = jnp.min(jnp.where(dist == m, lane, K), axis=1, keepdims=True)

        oh = (lane == idx).astype(jnp.float32)
        q = jax.lax.dot_general(w, oh, (((0,), (1,)), ((), ())),
                                preferred_element_type=jnp.float32)
        q_ref[0, :, T * a:T * (a + 1)] = q
        idx_ref[0, :, a:a + 1] = idx

        counts_ref[a:a + 1, :] += jnp.sum(oh, axis=0, keepdims=True)
        loss_ref[a] += jnp.sum(m)

    @pl.when(b == B - 1)
    def _finalize():
        p = counts_ref[...] / N
        ent = jnp.sum(p * jnp.log(p + 1e-10), axis=1)
        for a in range(A):
            perp_ref[a] = jnp.exp(-ent[a])


def _vq(x2, emb):
    return pl.pallas_call(
        _vq_body,
        grid=(B,),
        in_specs=[
            pl.BlockSpec((1, D, A * T), lambda b: (b, 0, 0)),
            pl.BlockSpec((A, K, D), lambda b: (0, 0, 0)),
        ],
        out_specs=[
            pl.BlockSpec((1, D, A * T), lambda b: (b, 0, 0)),
            pl.BlockSpec((1, T, A), lambda b: (b, 0, 0)),
            pl.BlockSpec(memory_space=pltpu.SMEM),
            pl.BlockSpec(memory_space=pltpu.SMEM),
        ],
        out_shape=[
            jax.ShapeDtypeStruct((B, D, A * T), jnp.float32),
            jax.ShapeDtypeStruct((B, T, A), jnp.int32),
            jax.ShapeDtypeStruct((A,), jnp.float32),
            jax.ShapeDtypeStruct((A,), jnp.float32),
        ],
        scratch_shapes=[
            pltpu.VMEM((A, K), jnp.float32),
        ],
    )(x2, emb)


def kernel(inputs, emb):
    x2 = inputs.reshape(B, D, A * T)
    q2, idx2, loss_sums, perps = _vq(x2, emb)
    quantized = q2.reshape(B, D, A, T)
    encoding_indices = idx2.reshape(N, A, 1)
    l = loss_sums / jnp.float32(N * D)
    q_loss = jnp.sum(l) / A
    e_loss = jnp.sum(0.25 * l) / A
    perplexity = jnp.sum(perps) / A
    return q_loss, e_loss, quantized, perplexity, encoding_indices
